# baseline (speedup 1.0000x reference)
import jax
import jax.numpy as jnp
import numpy as np
from jax import lax
from jax.experimental import pallas as pl
from jax.experimental.pallas import tpu as pltpu
from jax.experimental.pallas import tpu_sc as plsc

NUM_ADAPTERS = 16
RANK = 16
D_IN = 1024
D_OUT = 1024
TOPK = 4
TOKEN_TILE = 256
N_WORKERS = 32
LANES = 16
TOK_PER_W = 64
W_PER_TILE = TOKEN_TILE // TOK_PER_W


def _vn_kernel(x_ref, atf_ref, s_ref, n2t_ref):
    xb = x_ref[:].astype(jnp.bfloat16)
    V = jnp.dot(xb, atf_ref[:], preferred_element_type=jnp.float32)
    VV = V * V
    for w in range(W_PER_TILE):
        n2t_ref[w] = lax.dot_general(
            s_ref[:], VV[w * TOK_PER_W:(w + 1) * TOK_PER_W, :],
            dimension_numbers=(((0,), (1,)), ((), ())),
            preferred_element_type=jnp.float32)


def _max_tree(vs):
    while len(vs) > 1:
        vs = [jnp.maximum(vs[i], vs[i + 1]) for i in range(0, len(vs) - 1, 2)] \
            + ([vs[-1]] if len(vs) % 2 else [])
    return vs[0]


def _sc_routes(n2t_hbm, routes_hbm, n2_v, r_v):
    c = lax.axis_index("c")
    s = lax.axis_index("s")
    wid = s * 2 + c
    pltpu.sync_copy(n2t_hbm.at[wid], n2_v)
    for g in range(TOK_PER_W // LANES):
        sl = pl.ds(g * LANES, LANES)
        keys = []
        for a in range(NUM_ADAPTERS):
            row = n2_v[a, sl]
            k = (lax.bitcast_convert_type(row, jnp.int32) & -16) | (15 - a)
            keys.append(k)
        cur = keys
        for _ in range(TOPK - 1):
            m = _max_tree(cur)
            cur = [jnp.where(k == m, -1, k) for k in cur]
        thr = _max_tree(cur)
        for a in range(NUM_ADAPTERS):
            r_v[a, sl] = jnp.where(keys[a] >= thr, 1.0 / TOPK, 0.0)
    pltpu.sync_copy(r_v, routes_hbm.at[wid])


def _out_kernel(x_ref, wt_ref, bft_ref, bias_ref, atf_ref, rt_ref, st_ref,
                t_ref, t2_ref, out_ref):
    xb = x_ref[:].astype(jnp.bfloat16)
    V = jnp.dot(xb, atf_ref[:], preferred_element_type=jnp.float32).astype(
        jnp.bfloat16)
    M = jnp.concatenate(
        [lax.dot_general(rt_ref[w], st_ref[:],
                         dimension_numbers=(((0,), (0,)), ((), ())),
                         preferred_element_type=jnp.float32)
         for w in range(W_PER_TILE)], axis=0).astype(jnp.bfloat16)
    vs = jnp.dot(M * V, t_ref[:], preferred_element_type=jnp.float32)
    U = M * jnp.dot(vs.astype(jnp.bfloat16), t2_ref[:],
                    preferred_element_type=jnp.float32).astype(jnp.bfloat16)
    acc = jnp.dot(xb, wt_ref[:], preferred_element_type=jnp.float32)
    acc = acc + jnp.dot(U, bft_ref[:], preferred_element_type=jnp.float32)
    out_ref[:] = acc + bias_ref[:]


@jax.jit
def kernel(x, A, B, W, bias):
    b, s, _ = x.shape
    tokens = b * s
    n_workers = tokens // TOK_PER_W
    x2d = x.reshape(tokens, D_IN)
    atf = A.reshape(NUM_ADAPTERS * RANK, D_IN).T.astype(jnp.bfloat16)
    wt = W.T.astype(jnp.bfloat16)
    bft = B.transpose(0, 2, 1).reshape(NUM_ADAPTERS * RANK, D_OUT).astype(jnp.bfloat16)
    bias2d = bias.reshape(1, D_OUT)

    blk = np.zeros((NUM_ADAPTERS * RANK, NUM_ADAPTERS), dtype=np.float32)
    blk[np.arange(NUM_ADAPTERS * RANK), np.arange(NUM_ADAPTERS * RANK) // RANK] = 1.0
    rnk = np.zeros((NUM_ADAPTERS * RANK, RANK), dtype=np.float32)
    rnk[np.arange(NUM_ADAPTERS * RANK), np.arange(NUM_ADAPTERS * RANK) % RANK] = 1.0
    S = jnp.asarray(blk)
    St = jnp.asarray(blk.T)
    T = jnp.asarray(rnk, dtype=jnp.bfloat16)
    T2 = jnp.asarray(rnk.T, dtype=jnp.bfloat16)

    n_tiles = tokens // TOKEN_TILE
    const = lambda i: (0, 0)
    const3 = lambda i: (0, 0, 0)

    n2t = pl.pallas_call(
        _vn_kernel,
        grid=(n_tiles,),
        in_specs=[
            pl.BlockSpec((TOKEN_TILE, D_IN), lambda i: (i, 0)),
            pl.BlockSpec((D_IN, NUM_ADAPTERS * RANK), const),
            pl.BlockSpec((NUM_ADAPTERS * RANK, NUM_ADAPTERS), const),
        ],
        out_specs=pl.BlockSpec((W_PER_TILE, NUM_ADAPTERS, TOK_PER_W),
                               lambda i: (i, 0, 0)),
        out_shape=jax.ShapeDtypeStruct((n_workers, NUM_ADAPTERS, TOK_PER_W),
                                       jnp.float32),
    )(x2d, atf, S)

    routes_t = pl.kernel(
        _sc_routes,
        out_type=jax.ShapeDtypeStruct((n_workers, NUM_ADAPTERS, TOK_PER_W),
                                      jnp.float32),
        mesh=plsc.VectorSubcoreMesh(core_axis_name="c", subcore_axis_name="s",
                                    num_cores=2, num_subcores=16),
        scratch_types=[
            pltpu.VMEM((NUM_ADAPTERS, TOK_PER_W), jnp.float32),
            pltpu.VMEM((NUM_ADAPTERS, TOK_PER_W), jnp.float32),
        ],
    )(n2t)

    out = pl.pallas_call(
        _out_kernel,
        grid=(n_tiles,),
        in_specs=[
            pl.BlockSpec((TOKEN_TILE, D_IN), lambda i: (i, 0)),
            pl.BlockSpec((D_IN, D_OUT), const),
            pl.BlockSpec((NUM_ADAPTERS * RANK, D_OUT), const),
            pl.BlockSpec((1, D_OUT), const),
            pl.BlockSpec((D_IN, NUM_ADAPTERS * RANK), const),
            pl.BlockSpec((W_PER_TILE, NUM_ADAPTERS, TOK_PER_W),
                         lambda i: (i, 0, 0)),
            pl.BlockSpec((NUM_ADAPTERS, NUM_ADAPTERS * RANK), const),
            pl.BlockSpec((NUM_ADAPTERS * RANK, RANK), const),
            pl.BlockSpec((RANK, NUM_ADAPTERS * RANK), const),
        ],
        out_specs=pl.BlockSpec((TOKEN_TILE, D_OUT), lambda i: (i, 0)),
        out_shape=jax.ShapeDtypeStruct((tokens, D_OUT), jnp.float32),
    )(x2d, wt, bft, bias2d, atf, routes_t, St, T, T2)
    return out.reshape(b, s, D_OUT)


# speedup vs baseline: 1.1596x; 1.1596x over previous
import jax
import jax.numpy as jnp
import numpy as np
from jax import lax
from jax.experimental import pallas as pl
from jax.experimental.pallas import tpu as pltpu
from jax.experimental.pallas import tpu_sc as plsc

NUM_ADAPTERS = 16
RANK = 16
D_IN = 1024
D_OUT = 1024
TOPK = 4
TOKEN_TILE = 512
N_WORKERS = 32
LANES = 16
TOK_PER_W = 64
W_PER_TILE = TOKEN_TILE // TOK_PER_W


def _vn_kernel(x_ref, atf_ref, s_ref, v_ref, n2t_ref):
    xb = x_ref[:].astype(jnp.bfloat16)
    V = jnp.dot(xb, atf_ref[:], preferred_element_type=jnp.float32)
    v_ref[:] = V.astype(jnp.bfloat16)
    VV = V * V
    for w in range(W_PER_TILE):
        n2t_ref[w] = lax.dot_general(
            s_ref[:], VV[w * TOK_PER_W:(w + 1) * TOK_PER_W, :],
            dimension_numbers=(((0,), (1,)), ((), ())),
            preferred_element_type=jnp.float32)


def _max_tree(vs):
    while len(vs) > 1:
        vs = [jnp.maximum(vs[i], vs[i + 1]) for i in range(0, len(vs) - 1, 2)] \
            + ([vs[-1]] if len(vs) % 2 else [])
    return vs[0]


def _sc_routes(n2t_hbm, routes_hbm, n2_v, r_v):
    c = lax.axis_index("c")
    s = lax.axis_index("s")
    wid = s * 2 + c
    pltpu.sync_copy(n2t_hbm.at[wid], n2_v)
    for g in range(TOK_PER_W // LANES):
        sl = pl.ds(g * LANES, LANES)
        keys = []
        for a in range(NUM_ADAPTERS):
            row = n2_v[a, sl]
            k = (lax.bitcast_convert_type(row, jnp.int32) & -16) | (15 - a)
            keys.append(k)
        cur = keys
        for _ in range(TOPK - 1):
            m = _max_tree(cur)
            cur = [jnp.where(k == m, -1, k) for k in cur]
        thr = _max_tree(cur)
        for a in range(NUM_ADAPTERS):
            r_v[a, sl] = jnp.where(keys[a] >= thr, 1.0 / TOPK, 0.0)
    pltpu.sync_copy(r_v, routes_hbm.at[wid])


def _out_kernel(x_ref, wt_ref, bft_ref, bias_ref, v_ref, rt_ref, st_ref,
                t_ref, t2_ref, out_ref):
    xb = x_ref[:].astype(jnp.bfloat16)
    V = v_ref[:]
    M = jnp.concatenate(
        [lax.dot_general(rt_ref[w], st_ref[:],
                         dimension_numbers=(((0,), (0,)), ((), ())),
                         preferred_element_type=jnp.float32)
         for w in range(W_PER_TILE)], axis=0).astype(jnp.bfloat16)
    vs = jnp.dot(M * V, t_ref[:], preferred_element_type=jnp.float32)
    U = M * jnp.dot(vs.astype(jnp.bfloat16), t2_ref[:],
                    preferred_element_type=jnp.float32).astype(jnp.bfloat16)
    acc = jnp.dot(xb, wt_ref[:], preferred_element_type=jnp.float32)
    acc = acc + jnp.dot(U, bft_ref[:], preferred_element_type=jnp.float32)
    out_ref[:] = acc + bias_ref[:]


@jax.jit
def kernel(x, A, B, W, bias):
    b, s, _ = x.shape
    tokens = b * s
    n_workers = tokens // TOK_PER_W
    x2d = x.reshape(tokens, D_IN)
    atf = A.reshape(NUM_ADAPTERS * RANK, D_IN).T.astype(jnp.bfloat16)
    wt = W.T.astype(jnp.bfloat16)
    bft = B.transpose(0, 2, 1).reshape(NUM_ADAPTERS * RANK, D_OUT).astype(jnp.bfloat16)
    bias2d = bias.reshape(1, D_OUT)

    blk = np.zeros((NUM_ADAPTERS * RANK, NUM_ADAPTERS), dtype=np.float32)
    blk[np.arange(NUM_ADAPTERS * RANK), np.arange(NUM_ADAPTERS * RANK) // RANK] = 1.0
    rnk = np.zeros((NUM_ADAPTERS * RANK, RANK), dtype=np.float32)
    rnk[np.arange(NUM_ADAPTERS * RANK), np.arange(NUM_ADAPTERS * RANK) % RANK] = 1.0
    S = jnp.asarray(blk)
    St = jnp.asarray(blk.T)
    T = jnp.asarray(rnk, dtype=jnp.bfloat16)
    T2 = jnp.asarray(rnk.T, dtype=jnp.bfloat16)

    n_tiles = tokens // TOKEN_TILE
    const = lambda i: (0, 0)
    const3 = lambda i: (0, 0, 0)

    V, n2t = pl.pallas_call(
        _vn_kernel,
        grid=(n_tiles,),
        in_specs=[
            pl.BlockSpec((TOKEN_TILE, D_IN), lambda i: (i, 0)),
            pl.BlockSpec((D_IN, NUM_ADAPTERS * RANK), const),
            pl.BlockSpec((NUM_ADAPTERS * RANK, NUM_ADAPTERS), const),
        ],
        out_specs=[
            pl.BlockSpec((TOKEN_TILE, NUM_ADAPTERS * RANK), lambda i: (i, 0)),
            pl.BlockSpec((W_PER_TILE, NUM_ADAPTERS, TOK_PER_W),
                         lambda i: (i, 0, 0)),
        ],
        out_shape=[
            jax.ShapeDtypeStruct((tokens, NUM_ADAPTERS * RANK), jnp.bfloat16),
            jax.ShapeDtypeStruct((n_workers, NUM_ADAPTERS, TOK_PER_W),
                                 jnp.float32),
        ],
    )(x2d, atf, S)

    routes_t = pl.kernel(
        _sc_routes,
        out_type=jax.ShapeDtypeStruct((n_workers, NUM_ADAPTERS, TOK_PER_W),
                                      jnp.float32),
        mesh=plsc.VectorSubcoreMesh(core_axis_name="c", subcore_axis_name="s",
                                    num_cores=2, num_subcores=16),
        scratch_types=[
            pltpu.VMEM((NUM_ADAPTERS, TOK_PER_W), jnp.float32),
            pltpu.VMEM((NUM_ADAPTERS, TOK_PER_W), jnp.float32),
        ],
    )(n2t)

    out = pl.pallas_call(
        _out_kernel,
        grid=(n_tiles,),
        in_specs=[
            pl.BlockSpec((TOKEN_TILE, D_IN), lambda i: (i, 0)),
            pl.BlockSpec((D_IN, D_OUT), const),
            pl.BlockSpec((NUM_ADAPTERS * RANK, D_OUT), const),
            pl.BlockSpec((1, D_OUT), const),
            pl.BlockSpec((TOKEN_TILE, NUM_ADAPTERS * RANK), lambda i: (i, 0)),
            pl.BlockSpec((W_PER_TILE, NUM_ADAPTERS, TOK_PER_W),
                         lambda i: (i, 0, 0)),
            pl.BlockSpec((NUM_ADAPTERS, NUM_ADAPTERS * RANK), const),
            pl.BlockSpec((NUM_ADAPTERS * RANK, RANK), const),
            pl.BlockSpec((RANK, NUM_ADAPTERS * RANK), const),
        ],
        out_specs=pl.BlockSpec((TOKEN_TILE, D_OUT), lambda i: (i, 0)),
        out_shape=jax.ShapeDtypeStruct((tokens, D_OUT), jnp.float32),
    )(x2d, wt, bft, bias2d, V, routes_t, St, T, T2)
    return out.reshape(b, s, D_OUT)


# speedup vs baseline: 1.1708x; 1.0096x over previous
import jax
import jax.numpy as jnp
import numpy as np
from jax import lax
from jax.experimental import pallas as pl
from jax.experimental.pallas import tpu as pltpu
from jax.experimental.pallas import tpu_sc as plsc

NUM_ADAPTERS = 16
RANK = 16
D_IN = 1024
D_OUT = 1024
TOPK = 4
TOKEN_TILE = 1024
N_WORKERS = 32
LANES = 16
TOK_PER_W = 64
W_PER_TILE = TOKEN_TILE // TOK_PER_W


def _vn_kernel(x_ref, atf_ref, s_ref, v_ref, n2t_ref):
    xb = x_ref[:].astype(jnp.bfloat16)
    V = jnp.dot(xb, atf_ref[:], preferred_element_type=jnp.float32)
    v_ref[:] = V.astype(jnp.bfloat16)
    VV = V * V
    for w in range(W_PER_TILE):
        n2t_ref[w] = lax.dot_general(
            s_ref[:], VV[w * TOK_PER_W:(w + 1) * TOK_PER_W, :],
            dimension_numbers=(((0,), (1,)), ((), ())),
            preferred_element_type=jnp.float32)


def _max_tree(vs):
    while len(vs) > 1:
        vs = [jnp.maximum(vs[i], vs[i + 1]) for i in range(0, len(vs) - 1, 2)] \
            + ([vs[-1]] if len(vs) % 2 else [])
    return vs[0]


def _sc_routes(n2t_hbm, routes_hbm, n2_v, r_v):
    c = lax.axis_index("c")
    s = lax.axis_index("s")
    wid = s * 2 + c
    pltpu.sync_copy(n2t_hbm.at[wid], n2_v)
    for g in range(TOK_PER_W // LANES):
        sl = pl.ds(g * LANES, LANES)
        keys = []
        for a in range(NUM_ADAPTERS):
            row = n2_v[a, sl]
            k = (lax.bitcast_convert_type(row, jnp.int32) & -16) | (15 - a)
            keys.append(k)
        cur = keys
        for _ in range(TOPK - 1):
            m = _max_tree(cur)
            cur = [jnp.where(k == m, -1, k) for k in cur]
        thr = _max_tree(cur)
        for a in range(NUM_ADAPTERS):
            r_v[a, sl] = jnp.where(keys[a] >= thr, 1.0 / TOPK, 0.0)
    pltpu.sync_copy(r_v, routes_hbm.at[wid])


def _out_kernel(x_ref, wt_ref, bft_ref, bias_ref, v_ref, rt_ref, st_ref,
                t_ref, t2_ref, out_ref):
    xb = x_ref[:].astype(jnp.bfloat16)
    V = v_ref[:]
    M = jnp.concatenate(
        [lax.dot_general(rt_ref[w], st_ref[:],
                         dimension_numbers=(((0,), (0,)), ((), ())),
                         preferred_element_type=jnp.float32)
         for w in range(W_PER_TILE)], axis=0).astype(jnp.bfloat16)
    vs = jnp.dot(M * V, t_ref[:], preferred_element_type=jnp.float32)
    U = M * jnp.dot(vs.astype(jnp.bfloat16), t2_ref[:],
                    preferred_element_type=jnp.float32).astype(jnp.bfloat16)
    acc = jnp.dot(xb, wt_ref[:], preferred_element_type=jnp.float32)
    acc = acc + jnp.dot(U, bft_ref[:], preferred_element_type=jnp.float32)
    out_ref[:] = acc + bias_ref[:]


@jax.jit
def kernel(x, A, B, W, bias):
    b, s, _ = x.shape
    tokens = b * s
    n_workers = tokens // TOK_PER_W
    x2d = x.reshape(tokens, D_IN)
    atf = A.reshape(NUM_ADAPTERS * RANK, D_IN).T.astype(jnp.bfloat16)
    wt = W.T.astype(jnp.bfloat16)
    bft = B.transpose(0, 2, 1).reshape(NUM_ADAPTERS * RANK, D_OUT).astype(jnp.bfloat16)
    bias2d = bias.reshape(1, D_OUT)

    blk = np.zeros((NUM_ADAPTERS * RANK, NUM_ADAPTERS), dtype=np.float32)
    blk[np.arange(NUM_ADAPTERS * RANK), np.arange(NUM_ADAPTERS * RANK) // RANK] = 1.0
    rnk = np.zeros((NUM_ADAPTERS * RANK, RANK), dtype=np.float32)
    rnk[np.arange(NUM_ADAPTERS * RANK), np.arange(NUM_ADAPTERS * RANK) % RANK] = 1.0
    S = jnp.asarray(blk)
    St = jnp.asarray(blk.T)
    T = jnp.asarray(rnk, dtype=jnp.bfloat16)
    T2 = jnp.asarray(rnk.T, dtype=jnp.bfloat16)

    n_tiles = tokens // TOKEN_TILE
    const = lambda i: (0, 0)
    const3 = lambda i: (0, 0, 0)

    V, n2t = pl.pallas_call(
        _vn_kernel,
        grid=(n_tiles,),
        in_specs=[
            pl.BlockSpec((TOKEN_TILE, D_IN), lambda i: (i, 0)),
            pl.BlockSpec((D_IN, NUM_ADAPTERS * RANK), const),
            pl.BlockSpec((NUM_ADAPTERS * RANK, NUM_ADAPTERS), const),
        ],
        out_specs=[
            pl.BlockSpec((TOKEN_TILE, NUM_ADAPTERS * RANK), lambda i: (i, 0)),
            pl.BlockSpec((W_PER_TILE, NUM_ADAPTERS, TOK_PER_W),
                         lambda i: (i, 0, 0)),
        ],
        out_shape=[
            jax.ShapeDtypeStruct((tokens, NUM_ADAPTERS * RANK), jnp.bfloat16),
            jax.ShapeDtypeStruct((n_workers, NUM_ADAPTERS, TOK_PER_W),
                                 jnp.float32),
        ],
    )(x2d, atf, S)

    routes_t = pl.kernel(
        _sc_routes,
        out_type=jax.ShapeDtypeStruct((n_workers, NUM_ADAPTERS, TOK_PER_W),
                                      jnp.float32),
        mesh=plsc.VectorSubcoreMesh(core_axis_name="c", subcore_axis_name="s",
                                    num_cores=2, num_subcores=16),
        scratch_types=[
            pltpu.VMEM((NUM_ADAPTERS, TOK_PER_W), jnp.float32),
            pltpu.VMEM((NUM_ADAPTERS, TOK_PER_W), jnp.float32),
        ],
    )(n2t)

    out = pl.pallas_call(
        _out_kernel,
        grid=(n_tiles,),
        in_specs=[
            pl.BlockSpec((TOKEN_TILE, D_IN), lambda i: (i, 0)),
            pl.BlockSpec((D_IN, D_OUT), const),
            pl.BlockSpec((NUM_ADAPTERS * RANK, D_OUT), const),
            pl.BlockSpec((1, D_OUT), const),
            pl.BlockSpec((TOKEN_TILE, NUM_ADAPTERS * RANK), lambda i: (i, 0)),
            pl.BlockSpec((W_PER_TILE, NUM_ADAPTERS, TOK_PER_W),
                         lambda i: (i, 0, 0)),
            pl.BlockSpec((NUM_ADAPTERS, NUM_ADAPTERS * RANK), const),
            pl.BlockSpec((NUM_ADAPTERS * RANK, RANK), const),
            pl.BlockSpec((RANK, NUM_ADAPTERS * RANK), const),
        ],
        out_specs=pl.BlockSpec((TOKEN_TILE, D_OUT), lambda i: (i, 0)),
        out_shape=jax.ShapeDtypeStruct((tokens, D_OUT), jnp.float32),
    )(x2d, wt, bft, bias2d, V, routes_t, St, T, T2)
    return out.reshape(b, s, D_OUT)


# speedup vs baseline: 1.2323x; 1.0525x over previous
import jax
import jax.numpy as jnp
import numpy as np
from jax import lax
from jax.experimental import pallas as pl
from jax.experimental.pallas import tpu as pltpu
from jax.experimental.pallas import tpu_sc as plsc

NUM_ADAPTERS = 16
RANK = 16
D_IN = 1024
D_OUT = 1024
TOPK = 4
TOKEN_TILE = 1024
N_WORKERS = 32
LANES = 16
TOK_PER_W = 64
W_PER_TILE = TOKEN_TILE // TOK_PER_W


def _vn_kernel(x_ref, a2_ref, s_ref, v_ref, n2t_ref):
    xb = x_ref[:].astype(jnp.bfloat16)
    ab = a2_ref[:].astype(jnp.bfloat16)
    V = lax.dot_general(xb, ab, dimension_numbers=(((1,), (1,)), ((), ())),
                        preferred_element_type=jnp.float32)
    v_ref[:] = V.astype(jnp.bfloat16)
    VV = V * V
    for w in range(W_PER_TILE):
        n2t_ref[w] = lax.dot_general(
            s_ref[:], VV[w * TOK_PER_W:(w + 1) * TOK_PER_W, :],
            dimension_numbers=(((0,), (1,)), ((), ())),
            preferred_element_type=jnp.float32)


def _max_tree(vs):
    while len(vs) > 1:
        vs = [jnp.maximum(vs[i], vs[i + 1]) for i in range(0, len(vs) - 1, 2)] \
            + ([vs[-1]] if len(vs) % 2 else [])
    return vs[0]


def _sc_routes(n2t_hbm, routes_hbm, n2_v, r_v):
    c = lax.axis_index("c")
    s = lax.axis_index("s")
    wid = s * 2 + c
    pltpu.sync_copy(n2t_hbm.at[wid], n2_v)
    for g in range(TOK_PER_W // LANES):
        sl = pl.ds(g * LANES, LANES)
        keys = []
        for a in range(NUM_ADAPTERS):
            row = n2_v[a, sl]
            k = (lax.bitcast_convert_type(row, jnp.int32) & -16) | (15 - a)
            keys.append(k)
        cur = keys
        for _ in range(TOPK - 1):
            m = _max_tree(cur)
            cur = [jnp.where(k == m, -1, k) for k in cur]
        thr = _max_tree(cur)
        for a in range(NUM_ADAPTERS):
            r_v[a, sl] = jnp.where(keys[a] >= thr, 1.0 / TOPK, 0.0)
    pltpu.sync_copy(r_v, routes_hbm.at[wid])


def _out_kernel(x_ref, w_ref, bft_ref, bias_ref, v_ref, rt_ref, st_ref,
                t_ref, t2_ref, out_ref):
    xb = x_ref[:].astype(jnp.bfloat16)
    V = v_ref[:]
    M = jnp.concatenate(
        [lax.dot_general(rt_ref[w], st_ref[:],
                         dimension_numbers=(((0,), (0,)), ((), ())),
                         preferred_element_type=jnp.float32)
         for w in range(W_PER_TILE)], axis=0).astype(jnp.bfloat16)
    vs = jnp.dot(M * V, t_ref[:], preferred_element_type=jnp.float32)
    U = M * jnp.dot(vs.astype(jnp.bfloat16), t2_ref[:],
                    preferred_element_type=jnp.float32).astype(jnp.bfloat16)
    wb = w_ref[:].astype(jnp.bfloat16)
    acc = lax.dot_general(xb, wb, dimension_numbers=(((1,), (1,)), ((), ())),
                          preferred_element_type=jnp.float32)
    acc = acc + jnp.dot(U, bft_ref[:], preferred_element_type=jnp.float32)
    out_ref[:] = acc + bias_ref[:]


@jax.jit
def kernel(x, A, B, W, bias):
    b, s, _ = x.shape
    tokens = b * s
    n_workers = tokens // TOK_PER_W
    x2d = x.reshape(tokens, D_IN)
    a2 = A.reshape(NUM_ADAPTERS * RANK, D_IN)
    bft = B.transpose(0, 2, 1).reshape(NUM_ADAPTERS * RANK, D_OUT).astype(jnp.bfloat16)
    bias2d = bias.reshape(1, D_OUT)

    blk = np.zeros((NUM_ADAPTERS * RANK, NUM_ADAPTERS), dtype=np.float32)
    blk[np.arange(NUM_ADAPTERS * RANK), np.arange(NUM_ADAPTERS * RANK) // RANK] = 1.0
    rnk = np.zeros((NUM_ADAPTERS * RANK, RANK), dtype=np.float32)
    rnk[np.arange(NUM_ADAPTERS * RANK), np.arange(NUM_ADAPTERS * RANK) % RANK] = 1.0
    S = jnp.asarray(blk)
    St = jnp.asarray(blk.T)
    T = jnp.asarray(rnk, dtype=jnp.bfloat16)
    T2 = jnp.asarray(rnk.T, dtype=jnp.bfloat16)

    n_tiles = tokens // TOKEN_TILE
    const = lambda i: (0, 0)
    const3 = lambda i: (0, 0, 0)

    V, n2t = pl.pallas_call(
        _vn_kernel,
        grid=(n_tiles,),
        in_specs=[
            pl.BlockSpec((TOKEN_TILE, D_IN), lambda i: (i, 0)),
            pl.BlockSpec((NUM_ADAPTERS * RANK, D_IN), const),
            pl.BlockSpec((NUM_ADAPTERS * RANK, NUM_ADAPTERS), const),
        ],
        out_specs=[
            pl.BlockSpec((TOKEN_TILE, NUM_ADAPTERS * RANK), lambda i: (i, 0)),
            pl.BlockSpec((W_PER_TILE, NUM_ADAPTERS, TOK_PER_W),
                         lambda i: (i, 0, 0)),
        ],
        out_shape=[
            jax.ShapeDtypeStruct((tokens, NUM_ADAPTERS * RANK), jnp.bfloat16),
            jax.ShapeDtypeStruct((n_workers, NUM_ADAPTERS, TOK_PER_W),
                                 jnp.float32),
        ],
    )(x2d, a2, S)

    routes_t = pl.kernel(
        _sc_routes,
        out_type=jax.ShapeDtypeStruct((n_workers, NUM_ADAPTERS, TOK_PER_W),
                                      jnp.float32),
        mesh=plsc.VectorSubcoreMesh(core_axis_name="c", subcore_axis_name="s",
                                    num_cores=2, num_subcores=16),
        scratch_types=[
            pltpu.VMEM((NUM_ADAPTERS, TOK_PER_W), jnp.float32),
            pltpu.VMEM((NUM_ADAPTERS, TOK_PER_W), jnp.float32),
        ],
    )(n2t)

    out = pl.pallas_call(
        _out_kernel,
        grid=(n_tiles,),
        in_specs=[
            pl.BlockSpec((TOKEN_TILE, D_IN), lambda i: (i, 0)),
            pl.BlockSpec((D_OUT, D_IN), const),
            pl.BlockSpec((NUM_ADAPTERS * RANK, D_OUT), const),
            pl.BlockSpec((1, D_OUT), const),
            pl.BlockSpec((TOKEN_TILE, NUM_ADAPTERS * RANK), lambda i: (i, 0)),
            pl.BlockSpec((W_PER_TILE, NUM_ADAPTERS, TOK_PER_W),
                         lambda i: (i, 0, 0)),
            pl.BlockSpec((NUM_ADAPTERS, NUM_ADAPTERS * RANK), const),
            pl.BlockSpec((NUM_ADAPTERS * RANK, RANK), const),
            pl.BlockSpec((RANK, NUM_ADAPTERS * RANK), const),
        ],
        out_specs=pl.BlockSpec((TOKEN_TILE, D_OUT), lambda i: (i, 0)),
        out_shape=jax.ShapeDtypeStruct((tokens, D_OUT), jnp.float32),
    )(x2d, W, bft, bias2d, V, routes_t, St, T, T2)
    return out.reshape(b, s, D_OUT)
